# Initial kernel scaffold; baseline (speedup 1.0000x reference)
import jax
import jax.numpy as jnp
from jax.experimental import pallas as pl


def kernel(x, edge_index, Wb1, Wc1, bc1, b1, Wb2, Wc2, bc2, b2, Wb3, Wc3, bc3, b3):
    raise NotImplementedError("write your pallas kernel here")


# speedup vs baseline: 10.6305x; 10.6305x over previous
import functools

import jax
import jax.numpy as jnp
from jax import lax
from jax.experimental import pallas as pl
from jax.experimental.pallas import tpu as pltpu
from jax.experimental.pallas import tpu_sc as plsc

NUM_B = 4
C = 128
NW = 32
TILES = 16
BLK = 1024


def _sc_mesh():
    return plsc.VectorSubcoreMesh(core_axis_name="c", subcore_axis_name="s")


def _dot(a, b):
    return lax.dot_general(a, b, (((1,), (1,)), ((), ())),
                           preferred_element_type=jnp.float32)


def _deg_body(dstp, zeros, ones, out, dst_v, zer_v, one_v, accum):
    c = lax.axis_index("c")
    s = lax.axis_index("s")
    w = c * TILES + s
    nchunk = dst_v.shape[0]
    rows_per_tile = accum.shape[0] // TILES
    pltpu.sync_copy(dstp.at[w], dst_v)
    pltpu.sync_copy(zeros, zer_v)
    pltpu.sync_copy(ones, one_v)
    for t in range(rows_per_tile // C):
        pltpu.sync_copy(zer_v, accum.at[pl.ds(s * rows_per_tile + t * C, C)])
    plsc.subcore_barrier()

    def chunk(j, carry):
        pltpu.sync_copy(one_v, accum.at[dst_v.at[j]], add=True)
        return carry

    lax.fori_loop(0, nchunk, chunk, 0)
    plsc.subcore_barrier()
    pltpu.sync_copy(accum.at[pl.ds(s * rows_per_tile, rows_per_tile)],
                    out.at[c, pl.ds(s * rows_per_tile, rows_per_tile)])


def _sc_degree(dstp, np_pad):
    nchunk = dstp.shape[1]
    f = pl.kernel(
        _deg_body,
        out_type=jax.ShapeDtypeStruct((2, np_pad, 16), jnp.float32),
        mesh=_sc_mesh(),
        compiler_params=pltpu.CompilerParams(use_tc_tiling_on_sc=False),
        scratch_types=[
            pltpu.VMEM((nchunk, C), jnp.int32),
            pltpu.VMEM((C, 16), jnp.float32),
            pltpu.VMEM((C, 16), jnp.float32),
            pltpu.VMEM_SHARED((np_pad, 16), jnp.float32),
        ],
    )
    return f(dstp, jnp.zeros((C, 16), jnp.float32),
             jnp.ones((C, 16), jnp.float32))


def _spmm_body(table, srcp, dstp, zeros, out, src_v, dst_v, rows_v, zer_v,
               accum, sem):
    c = lax.axis_index("c")
    s = lax.axis_index("s")
    w = c * TILES + s
    nchunk = src_v.shape[0]
    rows_per_tile = accum.shape[0] // TILES
    pltpu.sync_copy(srcp.at[w], src_v)
    pltpu.sync_copy(dstp.at[w], dst_v)
    pltpu.sync_copy(zeros, zer_v)
    for t in range(rows_per_tile // C):
        pltpu.sync_copy(zer_v, accum.at[pl.ds(s * rows_per_tile + t * C, C)])
    plsc.subcore_barrier()

    def chunk(j, carry):
        pltpu.async_copy(table.at[src_v.at[j]], rows_v, sem).wait()
        pltpu.sync_copy(rows_v, accum.at[dst_v.at[j]], add=True)
        return carry

    lax.fori_loop(0, nchunk, chunk, 0)
    plsc.subcore_barrier()
    pltpu.sync_copy(accum.at[pl.ds(s * rows_per_tile, rows_per_tile)],
                    out.at[c, pl.ds(s * rows_per_tile, rows_per_tile)])


def _sc_spmm(table, srcp, dstp, tc_tiling=True):
    np_pad, dp = table.shape
    nchunk = srcp.shape[1]
    params = (None if tc_tiling
              else pltpu.CompilerParams(use_tc_tiling_on_sc=False))
    f = pl.kernel(
        _spmm_body,
        out_type=jax.ShapeDtypeStruct((2, np_pad, dp), jnp.float32),
        mesh=_sc_mesh(),
        compiler_params=params,
        scratch_types=[
            pltpu.VMEM((nchunk, C), jnp.int32),
            pltpu.VMEM((nchunk, C), jnp.int32),
            pltpu.VMEM((C, dp), jnp.float32),
            pltpu.VMEM((C, dp), jnp.float32),
            pltpu.VMEM_SHARED((np_pad, dp), jnp.float32),
            pltpu.SemaphoreType.DMA,
        ],
    )
    return f(table, srcp, dstp, jnp.zeros((C, dp), jnp.float32))


def _slice_pad(m, lo, w):
    hi = min(lo + w, m.shape[1])
    part = m[:, lo:hi]
    if hi - lo < w:
        part = jnp.concatenate(
            [part, jnp.zeros((m.shape[0], w - (hi - lo)), jnp.float32)], 1)
    return part


def _row_spec(cols):
    return pl.BlockSpec((BLK, cols), lambda i: (i, 0))


def _z_spec(cols):
    return pl.BlockSpec((2, BLK, cols), lambda i: (0, i, 0))


def _full_spec(shape):
    return pl.BlockSpec(shape, lambda i: tuple(0 for _ in shape))


def _bases_outputs(basesS, wts, w_o, chunk_os, widths):
    w_o[...] = wts
    lo = 0
    for o, w in zip(chunk_os, widths):
        o[...] = _slice_pad(basesS, lo, w)
        lo += w


def _tcA_body(deg0, deg1, x, Wb, Wc, bc, dinv_o, w_o, *chunk_os, widths):
    d = deg0[:, 0:1] + deg1[:, 0:1] + 1.0
    dinv = lax.rsqrt(d)
    dinv_o[...] = dinv
    basesS = _dot(x[...], Wb[...]) * dinv
    wts = _dot(x[...], Wc[...]) + bc[...]
    _bases_outputs(basesS, wts, w_o, chunk_os, widths)


def _tc_first(deg, xpad, Wb, Wc, bc, widths):
    np_pad, din = xpad.shape
    body = functools.partial(_tcA_body, widths=widths)
    return pl.pallas_call(
        body,
        grid=(np_pad // BLK,),
        in_specs=[
            _row_spec(16), _row_spec(16), _row_spec(din),
            _full_spec(Wb.shape), _full_spec(Wc.shape),
            _full_spec((1, NUM_B)),
        ],
        out_specs=[_row_spec(1), _row_spec(NUM_B)] +
                  [_row_spec(w) for w in widths],
        out_shape=[jax.ShapeDtypeStruct((np_pad, 1), jnp.float32),
                   jax.ShapeDtypeStruct((np_pad, NUM_B), jnp.float32)] +
                  [jax.ShapeDtypeStruct((np_pad, w), jnp.float32)
                   for w in widths],
    )(deg[0], deg[1], xpad, Wb, Wc, bc.reshape(1, NUM_B))


def _combine(zs, bSs, wts, dinv, bias, oc):
    aggs = [dinv * (z[0] + z[1] + bS) for z, bS in zip(zs, bSs)]
    agg = jnp.concatenate(aggs, axis=1) if len(aggs) > 1 else aggs[0]
    acc = wts[:, 0:1] * agg[:, 0:oc]
    for k in range(1, NUM_B):
        acc = acc + wts[:, k:k + 1] * agg[:, k * oc:(k + 1) * oc]
    return jnp.maximum(acc + bias, 0.0)


def _tcB_body(dinv, wts_p, bias_p, Wb, Wc, bc, *rest, nch, oc_p, widths):
    zs = [r[...] for r in rest[:nch]]
    bSs = [r[...] for r in rest[nch:2 * nch]]
    w_o = rest[2 * nch + 0]
    chunk_os = rest[2 * nch + 1:]
    dv = dinv[...]
    h = _combine(zs, bSs, wts_p[...], dv, bias_p[...], oc_p)
    basesS = _dot(h, Wb[...]) * dv
    wts = _dot(h, Wc[...]) + bc[...]
    _bases_outputs(basesS, wts, w_o, chunk_os, widths)


def _tc_mid(zs, bSs, wts_p, dinv, bias_p, Wb, Wc, bc, oc_p, widths):
    np_pad = dinv.shape[0]
    nch = len(zs)
    body = functools.partial(_tcB_body, nch=nch, oc_p=oc_p, widths=widths)
    zspecs = [_z_spec(z.shape[2]) for z in zs]
    bspecs = [_row_spec(b.shape[1]) for b in bSs]
    return pl.pallas_call(
        body,
        grid=(np_pad // BLK,),
        in_specs=[_row_spec(1), _row_spec(NUM_B), _full_spec((1, oc_p)),
                  _full_spec(Wb.shape), _full_spec(Wc.shape),
                  _full_spec((1, NUM_B))] + zspecs + bspecs,
        out_specs=[_row_spec(NUM_B)] + [_row_spec(w) for w in widths],
        out_shape=[jax.ShapeDtypeStruct((np_pad, NUM_B), jnp.float32)] +
                  [jax.ShapeDtypeStruct((np_pad, w), jnp.float32)
                   for w in widths],
    )(dinv, wts_p, bias_p.reshape(1, oc_p), Wb, Wc,
      bc.reshape(1, NUM_B), *zs, *bSs)


def _tcC_body(dinv, wts_p, bias_p, z, bS, out):
    out[...] = _combine([z[...]], [bS[...]], wts_p[...], dinv[...],
                        bias_p[...], 1)


def _tc_last(z, bS, wts_p, dinv, bias_p):
    np_pad = dinv.shape[0]
    return pl.pallas_call(
        _tcC_body,
        grid=(np_pad // BLK,),
        in_specs=[_row_spec(1), _row_spec(NUM_B), _full_spec((1, 1)),
                  _z_spec(z.shape[2]), _row_spec(bS.shape[1])],
        out_specs=pl.BlockSpec((BLK, 1), lambda i: (i, 0)),
        out_shape=jax.ShapeDtypeStruct((np_pad, 1), jnp.float32),
    )(dinv, wts_p, bias_p.reshape(1, 1), z, bS)


def _round_up(a, m):
    return (a + m - 1) // m * m


def kernel(x, edge_index, Wb1, Wc1, bc1, b1, Wb2, Wc2, bc2, b2, Wb3, Wc3,
           bc3, b3):
    n, din = x.shape
    e = edge_index.shape[1]
    np_pad = _round_up(n + 64, BLK * 2)
    n_pad_rows = np_pad - n
    ep = _round_up(e, NW * C)

    src, dst = edge_index[0], edge_index[1]
    pad_idx = (n + (jnp.arange(ep - e, dtype=jnp.int32) % n_pad_rows))
    srcp = jnp.concatenate([src, pad_idx]).reshape(NW, ep // (NW * C), C)
    dstp = jnp.concatenate([dst, pad_idx]).reshape(NW, ep // (NW * C), C)

    xpad = jnp.concatenate([x, jnp.zeros((n_pad_rows, din), x.dtype)])

    oc1 = Wb1.shape[0] // NUM_B
    oc2 = Wb2.shape[0] // NUM_B
    w1 = [C] * ((NUM_B * oc1 + C - 1) // C)
    w2 = [C] * ((NUM_B * oc2 + C - 1) // C)

    deg = _sc_degree(dstp, np_pad)
    dinv, wts1, *bS1 = _tc_first(deg, xpad, Wb1, Wc1, bc1, w1)
    z1 = [_sc_spmm(t, srcp, dstp) for t in bS1]
    wts2, *bS2 = _tc_mid(z1, bS1, wts1, dinv, b1, Wb2, Wc2, bc2, oc1, w2)
    z2 = [_sc_spmm(t, srcp, dstp) for t in bS2]
    wts3, *bS3 = _tc_mid(z2, bS2, wts2, dinv, b2, Wb3, Wc3, bc3, oc2, [16])
    z3 = _sc_spmm(bS3[0], srcp, dstp, tc_tiling=False)
    h3 = _tc_last(z3, bS3[0], wts3, dinv, b3)
    return h3[:n]
